# baseline (speedup 1.0000x reference)
import functools

import jax
import jax.numpy as jnp
from jax import lax
from jax.experimental import pallas as pl
from jax.experimental.pallas import tpu as pltpu
from jax.experimental.pallas import tpu_sc as plsc

_D = 64
_B = 4096
_H = 200
_N = _B * _H
_NC = 2
_NS = 16
_NW = _NC * _NS
_PER_W = _N // _NW
_CH = 128
_NCHUNK = _PER_W // _CH
_NBUF = 4
_NGROUP = _NCHUNK // _NBUF


def _gather_body(table, idx, out, idx_v, rows, *sems):
    gsem = sems[:_NBUF]
    psem = sems[_NBUF:]
    wid = lax.axis_index("s") * _NC + lax.axis_index("c")
    base = wid * _PER_W

    pltpu.sync_copy(idx.at[wid], idx_v)

    for b in range(_NBUF):
        pltpu.async_copy(table.at[idx_v.at[b]], rows.at[b], gsem[b])

    @pl.loop(0, _NGROUP - 1)
    def _group(g):
        for b in range(_NBUF):
            j = g * _NBUF + b
            o = out.at[pl.ds(base + j * _CH, _CH)]
            pltpu.make_async_copy(table.at[idx_v.at[j]], rows.at[b], gsem[b]).wait()
            pltpu.async_copy(rows.at[b], o, psem[b])
            pltpu.make_async_copy(rows.at[b], o, psem[b]).wait()
            pltpu.async_copy(table.at[idx_v.at[j + _NBUF]], rows.at[b], gsem[b])

    for b in range(_NBUF):
        j = (_NGROUP - 1) * _NBUF + b
        o = out.at[pl.ds(base + j * _CH, _CH)]
        pltpu.make_async_copy(table.at[idx_v.at[j]], rows.at[b], gsem[b]).wait()
        pltpu.async_copy(rows.at[b], o, psem[b])
    for b in range(_NBUF):
        j = (_NGROUP - 1) * _NBUF + b
        o = out.at[pl.ds(base + j * _CH, _CH)]
        pltpu.make_async_copy(rows.at[b], o, psem[b]).wait()


_mesh = plsc.VectorSubcoreMesh(core_axis_name="c", subcore_axis_name="s")

_gather = pl.kernel(
    _gather_body,
    out_type=jax.ShapeDtypeStruct((_N, _D), jnp.float32),
    mesh=_mesh,
    scratch_types=[
        pltpu.VMEM((_NCHUNK, _CH), jnp.int32),
        pltpu.VMEM((_NBUF, _CH, _D), jnp.float32),
    ] + [pltpu.SemaphoreType.DMA] * (2 * _NBUF),
    compiler_params=pltpu.CompilerParams(use_tc_tiling_on_sc=False),
)


@jax.jit
def kernel(x, embedding):
    idx = x.astype(jnp.int32).reshape(_NW, _NCHUNK, _CH)
    out = _gather(embedding, idx)
    return out.reshape(_B, _H, _D)
